# baseline (speedup 1.0000x reference)
import functools

import jax
import jax.numpy as jnp
import numpy as np
from jax import lax
from jax.experimental import pallas as pl
from jax.experimental.pallas import tpu as pltpu
from jax.experimental.pallas import tpu_sc as plsc

B, S, C = 4, 19, 192
H = W = 224
HG = WG = 56
OUT = 512

GSUB = 8 * W
WGP = 64
MC = 2 * WGP
NSLAB = 7
KBLK = NSLAB * GSUB
NSTEP = (H * W) // KBLK
PPAD = HG * WGP
NV = PPAD // 16

NTILE = 32
CGRP = NTILE // B
CPT = C // CGRP
CCH = 8
L = 16


def _pool_matrix() -> np.ndarray:
    j = np.arange(GSUB)
    pw = np.zeros((GSUB, MC), np.float32)
    pw[j, (j // W // 4) * WGP + (j % W) // 4] = 1.0
    return pw


def _pool_kernel(codes_ref, pw_ref, out_ref):
    x = codes_ref[0]
    for j in range(NSLAB):
        xj = x[:, j * GSUB:(j + 1) * GSUB]
        yp = jnp.dot(xj, pw_ref[...], preferred_element_type=jnp.float32)
        out_ref[0, 0, :, j * MC:(j + 1) * MC] = yp


def _sc_reduce(pooled_hbm, mask_hbm, sums_hbm, area_hbm,
               mask_v, rows_v, part_v, areap_v):
    wid = lax.axis_index("s") * 2 + lax.axis_index("c")
    cgrp = wid
    c0 = cgrp * CCH

    pltpu.sync_copy(mask_hbm, mask_v)

    @pl.when(cgrp < C // CCH)
    def _work():
        for h in range(NSTEP):
            pltpu.sync_copy(
                pooled_hbm.at[h, pl.ds(c0, CCH)],
                rows_v.at[:, pl.ds(h * (NSLAB * MC), NSLAB * MC)])
        for c in range(0, CCH, 2):
            def body(i, accs, c=c):
                off = i * L
                r0 = rows_v[c, pl.ds(off, L)]
                r1 = rows_v[c + 1, pl.ds(off, L)]
                out = []
                for s in range(S):
                    mv = mask_v[s, pl.ds(off, L)]
                    out.append(accs[2 * s] + r0 * mv)
                    out.append(accs[2 * s + 1] + r1 * mv)
                return tuple(out)
            accs = lax.fori_loop(
                0, NV, body,
                tuple(jnp.zeros((L,), jnp.float32) for _ in range(2 * S)))
            for s in range(S):
                part_v[c, pl.ds(s * L, L)] = accs[2 * s]
                part_v[c + 1, pl.ds(s * L, L)] = accs[2 * s + 1]
        pltpu.sync_copy(part_v, sums_hbm.at[pl.ds(c0, CCH)])

    @pl.when(cgrp == C // CCH)
    def _area():
        def abody(i, accs):
            off = i * L
            return tuple(accs[s] + mask_v[s, pl.ds(off, L)]
                         for s in range(S))
        aaccs = lax.fori_loop(
            0, NV, abody,
            tuple(jnp.zeros((L,), jnp.float32) for _ in range(S)))
        for s in range(S):
            areap_v[0, pl.ds(s * L, L)] = aaccs[s]
        pltpu.sync_copy(areap_v, area_hbm)


def _fin_kernel(sums16_ref, area16_ref, red_ref, fcw_ref, fcb_ref, out_ref):
    red = red_ref[...]
    sums = jnp.dot(sums16_ref[0], red,
                   preferred_element_type=jnp.float32)
    area = jnp.dot(area16_ref[0], red,
                   preferred_element_type=jnp.float32)
    inv = jnp.where(area > 0, 1.0 / (16.0 * jnp.maximum(area, 1.0)), 0.0)
    vec = sums * inv
    out_ref[0] = (jnp.dot(fcw_ref[...], vec,
                          preferred_element_type=jnp.float32)
                  + fcb_ref[...])


@jax.jit
def kernel(codes, segmap, fc_w, fc_b):
    codes3 = codes.reshape(B, C, H * W)
    pw = jnp.asarray(_pool_matrix())
    fcb2 = fc_b.reshape(OUT, 1)

    mask = jnp.pad(segmap.reshape(B, S, HG, WG),
                   ((0, 0), (0, 0), (0, 0), (0, WGP - WG))
                   ).reshape(B, S, PPAD)

    sc = functools.partial(
        pl.kernel,
        mesh=plsc.VectorSubcoreMesh(core_axis_name="c", subcore_axis_name="s"),
        out_type=(jax.ShapeDtypeStruct((C, S * L), jnp.float32),
                  jax.ShapeDtypeStruct((1, S * L), jnp.float32)),
        scratch_types=[
            pltpu.VMEM((S, PPAD), jnp.float32),
            pltpu.VMEM((CCH, PPAD), jnp.float32),
            pltpu.VMEM((CCH, S * L), jnp.float32),
            pltpu.VMEM((1, S * L), jnp.float32),
        ],
    )(_sc_reduce)

    sums_l, area_l = [], []
    for bb in range(B):
        pooled_b = pl.pallas_call(
            _pool_kernel,
            grid=(1, NSTEP),
            in_specs=[
                pl.BlockSpec((1, C, KBLK), lambda b, h, bb=bb: (bb, 0, h)),
                pl.BlockSpec((GSUB, MC), lambda b, h: (0, 0)),
            ],
            out_specs=pl.BlockSpec((1, 1, C, NSLAB * MC),
                                   lambda b, h: (b, h, 0, 0)),
            out_shape=jax.ShapeDtypeStruct((1, NSTEP, C, NSLAB * MC),
                                           jnp.float32),
        )(codes3, pw)
        s16, a16 = sc(pooled_b[0], mask[bb])
        sums_l.append(s16)
        area_l.append(a16)
    sums16 = jnp.stack(sums_l)
    area16 = jnp.stack(area_l)

    red = np.zeros((S * L, S), np.float32)
    red[np.arange(S * L), np.arange(S * L) // L] = 1.0
    red = jnp.asarray(red)

    out_t = pl.pallas_call(
        _fin_kernel,
        grid=(B,),
        in_specs=[
            pl.BlockSpec((1, C, S * L), lambda b: (b, 0, 0)),
            pl.BlockSpec((1, 1, S * L), lambda b: (b, 0, 0)),
            pl.BlockSpec((S * L, S), lambda b: (0, 0)),
            pl.BlockSpec((OUT, C), lambda b: (0, 0)),
            pl.BlockSpec((OUT, 1), lambda b: (0, 0)),
        ],
        out_specs=pl.BlockSpec((1, OUT, S), lambda b: (b, 0, 0)),
        out_shape=jax.ShapeDtypeStruct((B, OUT, S), jnp.float32),
    )(sums16, area16, red, fc_w, fcb2)
    return out_t.transpose(0, 2, 1)


# speedup vs baseline: 1.1536x; 1.1536x over previous
import functools

import jax
import jax.numpy as jnp
import numpy as np
from jax import lax
from jax.experimental import pallas as pl
from jax.experimental.pallas import tpu as pltpu
from jax.experimental.pallas import tpu_sc as plsc

B, S, C = 4, 19, 192
H = W = 224
HG = WG = 56
OUT = 512

GSUB = 8 * W
WGP = 64
MC = 2 * WGP
NSLAB = 7
KBLK = NSLAB * GSUB
NSTEP = (H * W) // KBLK
PPAD = HG * WGP
NV = PPAD // 16

NTILE = 32
CGRP = NTILE // B
CPT = C // CGRP
CCH = 8
L = 16


def _pool_matrix() -> np.ndarray:
    j = np.arange(GSUB)
    pw = np.zeros((GSUB, MC), np.float32)
    pw[j, (j // W // 4) * WGP + (j % W) // 4] = 1.0
    return pw


def _pool_kernel(codes_ref, pw_ref, out_ref):
    x = codes_ref[0]
    for j in range(NSLAB):
        xj = x[:, j * GSUB:(j + 1) * GSUB]
        yp = jnp.dot(xj, pw_ref[...], preferred_element_type=jnp.float32)
        out_ref[0, 0, :, j * MC:(j + 1) * MC] = yp


def _sc_reduce(pooled_hbm, mask_hbm, sums_hbm, area_hbm,
               mask_v, rows_v, part_v, areap_v):
    wid = lax.axis_index("s") * 2 + lax.axis_index("c")
    b = lax.rem(wid, B)
    cgrp = wid // B
    c0 = cgrp * CPT

    pltpu.sync_copy(mask_hbm.at[b], mask_v)

    for half in range(CPT // CCH):
        for h in range(NSTEP):
            pltpu.sync_copy(
                pooled_hbm.at[b, h, pl.ds(c0 + half * CCH, CCH)],
                rows_v.at[:, pl.ds(h * (NSLAB * MC), NSLAB * MC)])
        for c in range(0, CCH, 2):
            def body(i, accs, c=c):
                off = i * L
                r0 = rows_v[c, pl.ds(off, L)]
                r1 = rows_v[c + 1, pl.ds(off, L)]
                out = []
                for s in range(S):
                    mv = mask_v[s, pl.ds(off, L)]
                    out.append(accs[2 * s] + r0 * mv)
                    out.append(accs[2 * s + 1] + r1 * mv)
                return tuple(out)
            accs = lax.fori_loop(
                0, NV, body,
                tuple(jnp.zeros((L,), jnp.float32) for _ in range(2 * S)))
            for s in range(S):
                part_v[c, pl.ds(s * L, L)] = accs[2 * s]
                part_v[c + 1, pl.ds(s * L, L)] = accs[2 * s + 1]
        pltpu.sync_copy(part_v, sums_hbm.at[b, pl.ds(c0 + half * CCH, CCH)])

    @pl.when(cgrp == 0)
    def _area():
        def abody(i, accs):
            off = i * L
            return tuple(accs[s] + mask_v[s, pl.ds(off, L)]
                         for s in range(S))
        aaccs = lax.fori_loop(
            0, NV, abody,
            tuple(jnp.zeros((L,), jnp.float32) for _ in range(S)))
        for s in range(S):
            areap_v[0, pl.ds(s * L, L)] = aaccs[s]
        pltpu.sync_copy(areap_v, area_hbm.at[b])


def _fin_kernel(sums16_ref, area16_ref, red_ref, fcw_ref, fcb_ref, out_ref):
    red = red_ref[...]
    sums = jnp.dot(sums16_ref[0], red,
                   preferred_element_type=jnp.float32)
    area = jnp.dot(area16_ref[0], red,
                   preferred_element_type=jnp.float32)
    inv = jnp.where(area > 0, 1.0 / (16.0 * jnp.maximum(area, 1.0)), 0.0)
    vec = sums * inv
    out_ref[0] = (jnp.dot(fcw_ref[...], vec,
                          preferred_element_type=jnp.float32)
                  + fcb_ref[...])


@jax.jit
def kernel(codes, segmap, fc_w, fc_b):
    codes3 = codes.reshape(B, C, H * W)
    pw = jnp.asarray(_pool_matrix())
    fcb2 = fc_b.reshape(OUT, 1)

    pooled4 = pl.pallas_call(
        _pool_kernel,
        grid=(B, NSTEP),
        in_specs=[
            pl.BlockSpec((1, C, KBLK), lambda b, h: (b, 0, h)),
            pl.BlockSpec((GSUB, MC), lambda b, h: (0, 0)),
        ],
        out_specs=pl.BlockSpec((1, 1, C, NSLAB * MC), lambda b, h: (b, h, 0, 0)),
        out_shape=jax.ShapeDtypeStruct((B, NSTEP, C, NSLAB * MC), jnp.float32),
    )(codes3, pw)

    mask = jnp.pad(segmap.reshape(B, S, HG, WG),
                   ((0, 0), (0, 0), (0, 0), (0, WGP - WG))
                   ).reshape(B, S, PPAD)

    sc = functools.partial(
        pl.kernel,
        mesh=plsc.VectorSubcoreMesh(core_axis_name="c", subcore_axis_name="s"),
        out_type=(jax.ShapeDtypeStruct((B, C, S * L), jnp.float32),
                  jax.ShapeDtypeStruct((B, 1, S * L), jnp.float32)),
        scratch_types=[
            pltpu.VMEM((S, PPAD), jnp.float32),
            pltpu.VMEM((CCH, PPAD), jnp.float32),
            pltpu.VMEM((CCH, S * L), jnp.float32),
            pltpu.VMEM((1, S * L), jnp.float32),
        ],
    )(_sc_reduce)
    sums16, area16 = sc(pooled4, mask)

    red = np.zeros((S * L, S), np.float32)
    red[np.arange(S * L), np.arange(S * L) // L] = 1.0
    red = jnp.asarray(red)

    out_t = pl.pallas_call(
        _fin_kernel,
        grid=(B,),
        in_specs=[
            pl.BlockSpec((1, C, S * L), lambda b: (b, 0, 0)),
            pl.BlockSpec((1, 1, S * L), lambda b: (b, 0, 0)),
            pl.BlockSpec((S * L, S), lambda b: (0, 0)),
            pl.BlockSpec((OUT, C), lambda b: (0, 0)),
            pl.BlockSpec((OUT, 1), lambda b: (0, 0)),
        ],
        out_specs=pl.BlockSpec((1, OUT, S), lambda b: (b, 0, 0)),
        out_shape=jax.ShapeDtypeStruct((B, OUT, S), jnp.float32),
    )(sums16, area16, red, fc_w, fcb2)
    return out_t.transpose(0, 2, 1)
